# Initial kernel scaffold; baseline (speedup 1.0000x reference)
import jax
import jax.numpy as jnp
from jax.experimental import pallas as pl


def kernel(output, target):
    raise NotImplementedError("write your pallas kernel here")


# speedup vs baseline: 4.4368x; 4.4368x over previous
import jax
import jax.numpy as jnp
from jax.experimental import pallas as pl
from jax.experimental.pallas import tpu as pltpu

_SMOOTHING = 0.1
_N = 32000
_B = 2048
_CONF = 1.0 - _SMOOTHING
_SV = _SMOOTHING / (_N - 1)

_R = 256
_C = 3200
_NI = _B // _R
_NJ = _N // _C


def _stream_body(x_ref, t_ref, lse_ref, S_ref, cval_ref, m_ref, s_ref,
                 ci_ref, cv_ref):
    j = pl.program_id(1)
    x = x_ref[...]
    t = t_ref[...]

    @pl.when(j == 0)
    def _init():
        m_ref[...] = jnp.full((_R, 1), -jnp.inf, jnp.float32)
        s_ref[...] = jnp.zeros((_R, 1), jnp.float32)
        S_ref[...] = jnp.zeros((_R, 1), jnp.float32)
        ci_ref[...] = jnp.full((_R, 1), _N, jnp.int32)
        cv_ref[...] = x[:, 0:1]

    bm = jnp.max(x, axis=1, keepdims=True)
    m_old = m_ref[...]
    m_new = jnp.maximum(m_old, bm)
    p = jnp.exp(x - m_new)
    bs = jnp.sum(p, axis=1, keepdims=True)
    s_ref[...] = s_ref[...] * jnp.exp(m_old - m_new) + bs
    m_ref[...] = m_new
    S_ref[...] = S_ref[...] + jnp.sum(x, axis=1, keepdims=True)

    iota = jax.lax.broadcasted_iota(jnp.int32, (_R, _C), 1) + j * _C
    cand = jnp.where(t == 1, iota, _N)
    bidx = jnp.min(cand, axis=1, keepdims=True)
    bval = jnp.max(jnp.where(cand == bidx, x, -jnp.inf), axis=1, keepdims=True)
    take = bidx < ci_ref[...]
    ci_ref[...] = jnp.where(take, bidx, ci_ref[...])
    cv_ref[...] = jnp.where(take, bval, cv_ref[...])

    @pl.when(j == _NJ - 1)
    def _fin():
        lse_ref[...] = m_ref[...] + jnp.log(s_ref[...])
        cval_ref[...] = cv_ref[...]


def _combine_body(lse_ref, S_ref, cval_ref, out_ref):
    lse = lse_ref[...]
    S = S_ref[...]
    cval = cval_ref[...]
    rowloss = _SV * (_N * lse - S) + (_CONF - _SV) * (lse - cval)
    out_ref[...] = jnp.sum(rowloss, axis=(0, 1), keepdims=True) / _B


def kernel(output, target, interpret=False):
    lse, S, cval = pl.pallas_call(
        _stream_body,
        grid=(_NI, _NJ),
        in_specs=[
            pl.BlockSpec((_R, _C), lambda i, j: (i, j)),
            pl.BlockSpec((_R, _C), lambda i, j: (i, j)),
        ],
        out_specs=[
            pl.BlockSpec((_R, 1), lambda i, j: (i, 0)),
            pl.BlockSpec((_R, 1), lambda i, j: (i, 0)),
            pl.BlockSpec((_R, 1), lambda i, j: (i, 0)),
        ],
        out_shape=[
            jax.ShapeDtypeStruct((_B, 1), jnp.float32),
            jax.ShapeDtypeStruct((_B, 1), jnp.float32),
            jax.ShapeDtypeStruct((_B, 1), jnp.float32),
        ],
        scratch_shapes=[
            pltpu.VMEM((_R, 1), jnp.float32),
            pltpu.VMEM((_R, 1), jnp.float32),
            pltpu.VMEM((_R, 1), jnp.int32),
            pltpu.VMEM((_R, 1), jnp.float32),
        ],
        interpret=interpret,
    )(output, target)
    loss = pl.pallas_call(
        _combine_body,
        out_shape=jax.ShapeDtypeStruct((1, 1), jnp.float32),
        interpret=interpret,
    )(lse, S, cval)
    return loss[0, 0]


# speedup vs baseline: 6.8710x; 1.5486x over previous
import jax
import jax.numpy as jnp
from jax.experimental import pallas as pl
from jax.experimental.pallas import tpu as pltpu

_SMOOTHING = 0.1
_N = 32000
_B = 2048
_CONF = 1.0 - _SMOOTHING
_SV = _SMOOTHING / (_N - 1)

_R = 256
_C = 3200
_W = 640
_NI = _B // _R
_NJ = _N // _C


def _stream_body(x_ref, t_ref, lse_ref, S_ref, cval_ref, cidx_ref,
                 m_ref, s_ref):
    j = pl.program_id(1)
    x = x_ref[...]

    @pl.when(j == 0)
    def _init():
        t = t_ref[...]
        xw = x[:, :_W]
        iota = jax.lax.broadcasted_iota(jnp.int32, (_R, _W), 1)
        cand = jnp.where(t == 1, iota, _N)
        cidx = jnp.min(cand, axis=1, keepdims=True)
        cval = jnp.max(jnp.where(cand == cidx, xw, -jnp.inf), axis=1,
                       keepdims=True)
        cidx_ref[...] = cidx
        cval_ref[...] = jnp.where(cidx == _N, x[:, 0:1], cval)
        m_ref[...] = jnp.full((_R, 1), -jnp.inf, jnp.float32)
        s_ref[...] = jnp.zeros((_R, 1), jnp.float32)
        S_ref[...] = jnp.zeros((_R, 1), jnp.float32)

    bm = jnp.max(x, axis=1, keepdims=True)
    m_old = m_ref[...]
    m_new = jnp.maximum(m_old, bm)
    p = jnp.exp(x - m_new)
    bs = jnp.sum(p, axis=1, keepdims=True)
    s_ref[...] = s_ref[...] * jnp.exp(m_old - m_new) + bs
    m_ref[...] = m_new
    S_ref[...] = S_ref[...] + jnp.sum(x, axis=1, keepdims=True)

    @pl.when(j == _NJ - 1)
    def _fin():
        lse_ref[...] = m_ref[...] + jnp.log(s_ref[...])


def _fallback_body(x_ref, t_ref, cval_ref, ci_ref, cv_ref):
    j = pl.program_id(1)
    x = x_ref[...]
    t = t_ref[...]

    @pl.when(j == 0)
    def _init():
        ci_ref[...] = jnp.full((_R, 1), _N, jnp.int32)
        cv_ref[...] = x[:, 0:1]

    iota = jax.lax.broadcasted_iota(jnp.int32, (_R, _C), 1) + j * _C
    cand = jnp.where(t == 1, iota, _N)
    bidx = jnp.min(cand, axis=1, keepdims=True)
    bval = jnp.max(jnp.where(cand == bidx, x, -jnp.inf), axis=1, keepdims=True)
    take = bidx < ci_ref[...]
    ci_ref[...] = jnp.where(take, bidx, ci_ref[...])
    cv_ref[...] = jnp.where(take, bval, cv_ref[...])

    @pl.when(j == _NJ - 1)
    def _fin():
        cval_ref[...] = cv_ref[...]


def _combine_body(lse_ref, S_ref, cval_ref, out_ref):
    lse = lse_ref[...]
    S = S_ref[...]
    cval = cval_ref[...]
    rowloss = _SV * (_N * lse - S) + (_CONF - _SV) * (lse - cval)
    out_ref[...] = jnp.sum(rowloss, axis=(0, 1), keepdims=True) / _B


def _fallback_call(output, target, interpret):
    return pl.pallas_call(
        _fallback_body,
        grid=(_NI, _NJ),
        in_specs=[
            pl.BlockSpec((_R, _C), lambda i, j: (i, j)),
            pl.BlockSpec((_R, _C), lambda i, j: (i, j)),
        ],
        out_specs=pl.BlockSpec((_R, 1), lambda i, j: (i, 0)),
        out_shape=jax.ShapeDtypeStruct((_B, 1), jnp.float32),
        scratch_shapes=[
            pltpu.VMEM((_R, 1), jnp.int32),
            pltpu.VMEM((_R, 1), jnp.float32),
        ],
        interpret=interpret,
    )(output, target)


def kernel(output, target, interpret=False):
    lse, S, cval, cidx = pl.pallas_call(
        _stream_body,
        grid=(_NI, _NJ),
        in_specs=[
            pl.BlockSpec((_R, _C), lambda i, j: (i, j)),
            pl.BlockSpec((_R, _W), lambda i, j: (i, 0)),
        ],
        out_specs=[
            pl.BlockSpec((_R, 1), lambda i, j: (i, 0)),
            pl.BlockSpec((_R, 1), lambda i, j: (i, 0)),
            pl.BlockSpec((_R, 1), lambda i, j: (i, 0)),
            pl.BlockSpec((_R, 1), lambda i, j: (i, 0)),
        ],
        out_shape=[
            jax.ShapeDtypeStruct((_B, 1), jnp.float32),
            jax.ShapeDtypeStruct((_B, 1), jnp.float32),
            jax.ShapeDtypeStruct((_B, 1), jnp.float32),
            jax.ShapeDtypeStruct((_B, 1), jnp.int32),
        ],
        scratch_shapes=[
            pltpu.VMEM((_R, 1), jnp.float32),
            pltpu.VMEM((_R, 1), jnp.float32),
        ],
        interpret=interpret,
    )(output, target)

    unresolved = jnp.any(cidx == _N)
    cval = jax.lax.cond(
        unresolved,
        lambda o, t, cv: _fallback_call(o, t, interpret),
        lambda o, t, cv: cv,
        output, target, cval,
    )

    loss = pl.pallas_call(
        _combine_body,
        out_shape=jax.ShapeDtypeStruct((1, 1), jnp.float32),
        interpret=interpret,
    )(lse, S, cval)
    return loss[0, 0]


# speedup vs baseline: 9.5708x; 1.3929x over previous
import jax
import jax.numpy as jnp
from jax.experimental import pallas as pl
from jax.experimental.pallas import tpu as pltpu

_SMOOTHING = 0.1
_N = 32000
_B = 2048
_CONF = 1.0 - _SMOOTHING
_SV = _SMOOTHING / (_N - 1)

_R = 64
_W = 640
_NI = _B // _R

_RF = 256
_CF = 3200
_NIF = _B // _RF
_NJF = _N // _CF

_L = 128
_NK = _N // _L
_LOG2E = 1.4426950408889634


def _stream_body(x_ref, t_ref, lse_ref, S_ref, cval_ref, cidx_ref):
    t = t_ref[...]
    xw = x_ref[:, :_W]
    iota = jax.lax.broadcasted_iota(jnp.int32, (_R, _W), 1)
    cand = jnp.where(t == 1, iota, _N)
    cidx = jnp.min(cand, axis=1, keepdims=True)
    cval = jnp.max(jnp.where(cand == cidx, xw, -jnp.inf), axis=1,
                   keepdims=True)
    cidx_ref[...] = cidx
    cval_ref[...] = jnp.where(cidx == _N, x_ref[:, 0:1], cval)

    chunk0 = x_ref[:, 0:_L]
    m = chunk0
    Ss = chunk0
    for k in range(1, _NK):
        xk = x_ref[:, k * _L:(k + 1) * _L]
        m = jnp.maximum(m, xk)
        Ss = Ss + xk
    mm = m * _LOG2E
    s = jnp.exp2(chunk0 * _LOG2E - mm)
    for k in range(1, _NK):
        s = s + jnp.exp2(x_ref[:, k * _L:(k + 1) * _L] * _LOG2E - mm)

    m_row = jnp.max(m, axis=1, keepdims=True)
    s_row = jnp.sum(s * jnp.exp(m - m_row), axis=1, keepdims=True)
    lse_ref[...] = m_row + jnp.log(s_row)
    S_ref[...] = jnp.sum(Ss, axis=1, keepdims=True)


def _fallback_body(x_ref, t_ref, cval_ref, ci_ref, cv_ref):
    j = pl.program_id(1)
    x = x_ref[...]
    t = t_ref[...]

    @pl.when(j == 0)
    def _init():
        ci_ref[...] = jnp.full((_RF, 1), _N, jnp.int32)
        cv_ref[...] = x[:, 0:1]

    iota = jax.lax.broadcasted_iota(jnp.int32, (_RF, _CF), 1) + j * _CF
    cand = jnp.where(t == 1, iota, _N)
    bidx = jnp.min(cand, axis=1, keepdims=True)
    bval = jnp.max(jnp.where(cand == bidx, x, -jnp.inf), axis=1, keepdims=True)
    take = bidx < ci_ref[...]
    ci_ref[...] = jnp.where(take, bidx, ci_ref[...])
    cv_ref[...] = jnp.where(take, bval, cv_ref[...])

    @pl.when(j == _NJF - 1)
    def _fin():
        cval_ref[...] = cv_ref[...]


def _combine_body(lse_ref, S_ref, cval_ref, out_ref):
    lse = lse_ref[...]
    S = S_ref[...]
    cval = cval_ref[...]
    rowloss = _SV * (_N * lse - S) + (_CONF - _SV) * (lse - cval)
    out_ref[...] = jnp.sum(rowloss, axis=(0, 1), keepdims=True) / _B


def _fallback_call(output, target, interpret):
    return pl.pallas_call(
        _fallback_body,
        grid=(_NIF, _NJF),
        in_specs=[
            pl.BlockSpec((_RF, _CF), lambda i, j: (i, j)),
            pl.BlockSpec((_RF, _CF), lambda i, j: (i, j)),
        ],
        out_specs=pl.BlockSpec((_RF, 1), lambda i, j: (i, 0)),
        out_shape=jax.ShapeDtypeStruct((_B, 1), jnp.float32),
        scratch_shapes=[
            pltpu.VMEM((_RF, 1), jnp.int32),
            pltpu.VMEM((_RF, 1), jnp.float32),
        ],
        interpret=interpret,
    )(output, target)


def kernel(output, target, interpret=False):
    lse, S, cval, cidx = pl.pallas_call(
        _stream_body,
        grid=(_NI,),
        in_specs=[
            pl.BlockSpec((_R, _N), lambda i: (i, 0)),
            pl.BlockSpec((_R, _W), lambda i: (i, 0)),
        ],
        out_specs=[
            pl.BlockSpec((_R, 1), lambda i: (i, 0)),
            pl.BlockSpec((_R, 1), lambda i: (i, 0)),
            pl.BlockSpec((_R, 1), lambda i: (i, 0)),
            pl.BlockSpec((_R, 1), lambda i: (i, 0)),
        ],
        out_shape=[
            jax.ShapeDtypeStruct((_B, 1), jnp.float32),
            jax.ShapeDtypeStruct((_B, 1), jnp.float32),
            jax.ShapeDtypeStruct((_B, 1), jnp.float32),
            jax.ShapeDtypeStruct((_B, 1), jnp.int32),
        ],
        interpret=interpret,
    )(output, target)

    unresolved = jnp.any(cidx == _N)
    cval = jax.lax.cond(
        unresolved,
        lambda o, t, cv: _fallback_call(o, t, interpret),
        lambda o, t, cv: cv,
        output, target, cval,
    )

    loss = pl.pallas_call(
        _combine_body,
        out_shape=jax.ShapeDtypeStruct((1, 1), jnp.float32),
        interpret=interpret,
    )(lse, S, cval)
    return loss[0, 0]
